# baseline (speedup 1.0000x reference)
import jax
import jax.numpy as jnp
from jax.experimental import pallas as pl

_BLK = 8192


def _body(x_ref, ew_ref, eb_ref, w1t_ref, b1_ref, w2t_ref, b2_ref,
          w3t_ref, b3_ref, mu_ref, o_ref):
    f32 = jnp.float32
    v1 = jnp.dot(ew_ref[...], w1t_ref[...], preferred_element_type=f32)
    c1 = jnp.dot(eb_ref[...], w1t_ref[...], preferred_element_type=f32) + b1_ref[...]
    xv = x_ref[...]
    h = jnp.maximum(xv * v1 + c1, 0.0)
    h = jnp.dot(h, w2t_ref[...], preferred_element_type=f32) + b2_ref[...]
    h = jnp.maximum(h, 0.0)
    h = jnp.dot(h, w3t_ref[...], preferred_element_type=f32) + b3_ref[...]
    m = jnp.max(h, axis=1, keepdims=True)
    e = jnp.exp(h - m)
    s = jnp.sum(e, axis=1, keepdims=True)
    num = jnp.sum(e * mu_ref[...], axis=1, keepdims=True)
    o_ref[...] = num / s


def kernel(x, enc_w, enc_b, W1, b1, W2, b2, W3, b3, mu_proj):
    B = x.shape[0]
    grid = (B // _BLK,)
    w1t = W1.T
    w2t = W2.T
    w3t = W3.T
    b1r = b1.reshape(1, -1)
    b2r = b2.reshape(1, -1)
    b3r = b3.reshape(1, -1)
    mur = mu_proj.reshape(1, -1)

    full = lambda shp: pl.BlockSpec(shp, lambda i: (0, 0))
    out = pl.pallas_call(
        _body,
        grid=grid,
        in_specs=[
            pl.BlockSpec((_BLK, 1), lambda i: (i, 0)),
            full(enc_w.shape),
            full(enc_b.shape),
            full(w1t.shape), full(b1r.shape),
            full(w2t.shape), full(b2r.shape),
            full(w3t.shape), full(b3r.shape),
            full(mur.shape),
        ],
        out_specs=pl.BlockSpec((_BLK, 1), lambda i: (i, 0)),
        out_shape=jax.ShapeDtypeStruct((B, 1), jnp.float32),
    )(x, enc_w, enc_b, w1t, b1r, w2t, b2r, w3t, b3r, mur)
    return out


# speedup vs baseline: 1.1014x; 1.1014x over previous
import jax
import jax.numpy as jnp
from jax.experimental import pallas as pl

_BLK = 8192
_LOG2E = 1.4426950408889634


def _body(x_ref, ew_ref, eb_ref, w1t_ref, b1_ref, w2t_ref, b2_ref,
          a_ref, b3c_ref, p_ref, o_ref):
    f32 = jnp.float32
    v1 = jnp.dot(ew_ref[...], w1t_ref[...], preferred_element_type=f32)
    c1 = jnp.dot(eb_ref[...], w1t_ref[...], preferred_element_type=f32) + b1_ref[...]
    xv = x_ref[...]
    h = jnp.maximum(xv * v1 + c1, 0.0)
    h = jnp.dot(h, w2t_ref[...], preferred_element_type=f32) + b2_ref[...]
    h = jnp.maximum(h, 0.0)
    g = jnp.dot(h, a_ref[...], preferred_element_type=f32)
    l = g[:, :128] - g[:, 128:129] + b3c_ref[...]
    e = jnp.exp2(jnp.maximum(l, -100.0))
    r = jnp.dot(e, p_ref[...], preferred_element_type=f32)
    o_ref[...] = r[:, 0:1] / r[:, 1:2]


def kernel(x, enc_w, enc_b, W1, b1, W2, b2, W3, b3, mu_proj):
    B = x.shape[0]
    N = enc_w.shape[1]
    grid = (B // _BLK,)
    w1t = W1.T
    w2t = W2.T
    b1r = b1.reshape(1, N)
    b2r = b2.reshape(1, N)
    w3s = W3.T * _LOG2E
    b3s = (b3 * _LOG2E).reshape(1, N)
    u = jnp.max(w3s, axis=1, keepdims=True)
    a = jnp.concatenate([w3s, u, jnp.zeros((N, N - 1), jnp.float32)], axis=1)
    b3c = b3s - jnp.max(b3s)
    p = jnp.concatenate([mu_proj, jnp.ones((N, 1), jnp.float32)], axis=1)

    full = lambda shp: pl.BlockSpec(shp, lambda i: (0, 0))
    out = pl.pallas_call(
        _body,
        grid=grid,
        in_specs=[
            pl.BlockSpec((_BLK, 1), lambda i: (i, 0)),
            full(enc_w.shape),
            full(enc_b.shape),
            full(w1t.shape), full(b1r.shape),
            full(w2t.shape), full(b2r.shape),
            full(a.shape), full(b3c.shape),
            full(p.shape),
        ],
        out_specs=pl.BlockSpec((_BLK, 1), lambda i: (i, 0)),
        out_shape=jax.ShapeDtypeStruct((B, 1), jnp.float32),
    )(x, enc_w, enc_b, w1t, b1r, w2t, b2r, a, b3c, p)
    return out


# speedup vs baseline: 1.1038x; 1.0022x over previous
import jax
import jax.numpy as jnp
from jax import lax
from jax.experimental import pallas as pl

_BLK = 8192
_LOG2E = 1.4426950408889634
_N = 128


def _dot_t(a, b):
    return lax.dot_general(a, b, (((1,), (1,)), ((), ())),
                           preferred_element_type=jnp.float32)


def _body(x_ref, ew_ref, eb_ref, w1_ref, b1_ref, w2_ref, b2_ref,
          w3_ref, b3_ref, mu_ref, o_ref):
    f32 = jnp.float32
    w1 = w1_ref[...]
    v1 = _dot_t(ew_ref[...], w1)
    c1 = _dot_t(eb_ref[...], w1) + b1_ref[...]
    w3m = w3_ref[...] * _LOG2E
    b3m = b3_ref[...] * _LOG2E
    b3c = b3m - jnp.max(b3m)
    u = jnp.max(w3m, axis=0, keepdims=True)
    ub = jnp.broadcast_to(u.reshape(_N, 1), (_N, _N))
    mub = jnp.broadcast_to(mu_ref[...], (_N, _N))
    one = jnp.ones((_N, _N), f32)

    xv = x_ref[...]
    h = jnp.dot(xv, v1, preferred_element_type=f32) + c1
    h = jnp.maximum(h, 0.0)
    h = _dot_t(h, w2_ref[...]) + b2_ref[...]
    h = jnp.maximum(h, 0.0)
    l = _dot_t(h, w3m)
    m = jnp.dot(h, ub, preferred_element_type=f32)
    e = jnp.exp2(jnp.maximum(l - m + b3c, -100.0))
    num = jnp.dot(e, mub, preferred_element_type=f32)
    den = jnp.dot(e, one, preferred_element_type=f32)
    o_ref[...] = num[:, 0:1] / den[:, 0:1]


def kernel(x, enc_w, enc_b, W1, b1, W2, b2, W3, b3, mu_proj):
    B = x.shape[0]
    N = enc_w.shape[1]
    grid = (B // _BLK,)
    b1r = b1.reshape(1, N)
    b2r = b2.reshape(1, N)
    b3r = b3.reshape(1, N)

    full = lambda shp: pl.BlockSpec(shp, lambda i: (0, 0))
    out = pl.pallas_call(
        _body,
        grid=grid,
        in_specs=[
            pl.BlockSpec((_BLK, 1), lambda i: (i, 0)),
            full(enc_w.shape),
            full(enc_b.shape),
            full(W1.shape), full(b1r.shape),
            full(W2.shape), full(b2r.shape),
            full(W3.shape), full(b3r.shape),
            full(mu_proj.shape),
        ],
        out_specs=pl.BlockSpec((_BLK, 1), lambda i: (i, 0)),
        out_shape=jax.ShapeDtypeStruct((B, 1), jnp.float32),
    )(x, enc_w, enc_b, W1, b1r, W2, b2r, W3, b3r, mu_proj)
    return out


# speedup vs baseline: 3.0476x; 2.7609x over previous
import jax
import jax.numpy as jnp
from jax.experimental import pallas as pl

_S = 8192
_LOG2E = 1.4426950408889634
_N = 128


def _body(x_ref, ew_ref, eb_ref, w1_ref, b1_ref, w2_ref, b2_ref,
          w3_ref, b3_ref, mu_ref, o_ref):
    f32 = jnp.float32
    w1 = w1_ref[...]
    v1 = jnp.dot(w1, ew_ref[...], preferred_element_type=f32)
    c1 = jnp.dot(w1, eb_ref[...], preferred_element_type=f32) + b1_ref[...]
    w3m = w3_ref[...] * _LOG2E
    b3m = b3_ref[...] * _LOG2E
    b3c = b3m - jnp.max(b3m)
    u = jnp.max(w3m, axis=0, keepdims=True)
    p2 = jnp.concatenate([mu_ref[...].reshape(1, _N),
                          jnp.ones((1, _N), f32)], axis=0)

    xr = x_ref[...].reshape(1, _S)
    h = jnp.dot(v1, xr, preferred_element_type=f32) + c1
    h = jnp.maximum(h, 0.0)
    h = jnp.dot(w2_ref[...], h, preferred_element_type=f32) + b2_ref[...]
    h = jnp.maximum(h, 0.0)
    l = jnp.dot(w3m, h, preferred_element_type=f32)
    m = jnp.dot(u, h, preferred_element_type=f32)
    e = jnp.exp2(jnp.maximum(l + b3c - m, -100.0))
    r = jnp.dot(p2, e, preferred_element_type=f32)
    o_ref[...] = (r[0:1, :] / r[1:2, :]).reshape(1, 1, _S)


def kernel(x, enc_w, enc_b, W1, b1, W2, b2, W3, b3, mu_proj):
    B = x.shape[0]
    N = enc_w.shape[1]
    grid = (B // _S,)
    x3 = x.reshape(B // _S, 1, _S)
    ewc = enc_w.reshape(N, 1)
    ebc = enc_b.reshape(N, 1)
    b1c = b1.reshape(N, 1)
    b2c = b2.reshape(N, 1)
    b3c = b3.reshape(N, 1)

    full = lambda shp: pl.BlockSpec(shp, lambda i: tuple(0 for _ in shp))
    out = pl.pallas_call(
        _body,
        grid=grid,
        in_specs=[
            pl.BlockSpec((1, 1, _S), lambda i: (i, 0, 0)),
            full(ewc.shape),
            full(ebc.shape),
            full(W1.shape), full(b1c.shape),
            full(W2.shape), full(b2c.shape),
            full(W3.shape), full(b3c.shape),
            full(mu_proj.shape),
        ],
        out_specs=pl.BlockSpec((1, 1, _S), lambda i: (i, 0, 0)),
        out_shape=jax.ShapeDtypeStruct((B // _S, 1, _S), jnp.float32),
    )(x3, ewc, ebc, W1, b1c, W2, b2c, W3, b3c, mu_proj)
    return out.reshape(B, 1)
